# baseline (speedup 1.0000x reference)
import functools

import jax
import jax.numpy as jnp
from jax.experimental import pallas as pl
from jax.experimental.pallas import tpu as pltpu

_LOG2E = 1.4426950408889634
_NOUT = 11
_R = 256
_RC = 16


def _histo_body(k2_ref, b2_ref, x_ref, o_ref):
    def chunk(i, carry):
        rows = pl.ds(i * _RC, _RC)
        ys = [x_ref[c, rows, :] for c in range(3)]
        for j in range(_NOUT):
            kj = k2_ref[j]
            bj = b2_ref[j]
            acc = None
            for y in ys:
                x2 = y * kj + bj
                t = jnp.exp2(jnp.minimum(x2, -x2))
                s = (t + t) / (1.0 + t * t)
                acc = s if acc is None else acc + s
            o_ref[rows, j * 512:(j + 1) * 512] = acc
        return carry

    jax.lax.fori_loop(0, _R // _RC, chunk, 0)


@functools.partial(jax.jit, static_argnames=())
def kernel(image, kernel, bias):
    B, H, W, C = image.shape
    xt = jnp.transpose(image, (3, 0, 1, 2)).reshape(C, B * H, W)
    k2 = (kernel * _LOG2E).astype(jnp.float32)
    b2 = (bias * _LOG2E).astype(jnp.float32)

    out = pl.pallas_call(
        _histo_body,
        out_shape=jax.ShapeDtypeStruct((B * H, _NOUT * W), jnp.float32),
        grid=(B * H // _R,),
        in_specs=[
            pl.BlockSpec(memory_space=pltpu.SMEM),
            pl.BlockSpec(memory_space=pltpu.SMEM),
            pl.BlockSpec((C, _R, W), lambda i: (0, i, 0)),
        ],
        out_specs=pl.BlockSpec((_R, _NOUT * W), lambda i: (i, 0)),
        compiler_params=pltpu.CompilerParams(
            dimension_semantics=("arbitrary",),
            vmem_limit_bytes=56 * 1024 * 1024,
        ),
        name="histo_sech",
    )(k2, b2, xt)

    return out.reshape(B, H, W, _NOUT)


# speedup vs baseline: 1.5223x; 1.5223x over previous
import functools

import jax
import jax.numpy as jnp
import numpy as np
from jax.experimental import pallas as pl
from jax.experimental.pallas import tpu as pltpu

_LOG2E = 1.4426950408889634
_NOUT = 11
_R = 256
_RC = 16
_SIGN = np.uint32(0x80000000)


def _histo_body(k2_ref, b2_ref, x_ref, o_ref):
    def chunk(i, carry):
        rows = pl.ds(i * _RC, _RC)
        ys = [x_ref[c, rows, :].astype(jnp.float32) for c in range(3)]
        for j in range(_NOUT):
            kj = k2_ref[j]
            bj = b2_ref[j]
            acc = None
            for y in ys:
                x2 = y * kj + bj
                neg_abs = pltpu.bitcast(
                    pltpu.bitcast(x2, jnp.uint32) | _SIGN, jnp.float32)
                t = jnp.exp2(neg_abs)
                s = t * (1.0 / (1.0 + t * t))
                acc = s if acc is None else acc + s
            o_ref[j, rows, :] = (acc + acc).astype(jnp.bfloat16)
        return carry

    jax.lax.fori_loop(0, _R // _RC, chunk, 0)


@functools.partial(jax.jit, static_argnames=())
def kernel(image, kernel, bias):
    B, H, W, C = image.shape
    xt = jnp.transpose(image, (3, 0, 1, 2)).reshape(C, B * H, W).astype(
        jnp.bfloat16)
    k2 = (kernel * _LOG2E).astype(jnp.float32)
    b2 = (bias * _LOG2E).astype(jnp.float32)

    out = pl.pallas_call(
        _histo_body,
        out_shape=jax.ShapeDtypeStruct((_NOUT, B * H, W), jnp.bfloat16),
        grid=(B * H // _R,),
        in_specs=[
            pl.BlockSpec(memory_space=pltpu.SMEM),
            pl.BlockSpec(memory_space=pltpu.SMEM),
            pl.BlockSpec((C, _R, W), lambda i: (0, i, 0)),
        ],
        out_specs=pl.BlockSpec((_NOUT, _R, W), lambda i: (0, i, 0)),
        compiler_params=pltpu.CompilerParams(
            dimension_semantics=("arbitrary",),
            vmem_limit_bytes=56 * 1024 * 1024,
        ),
        name="histo_sech",
    )(k2, b2, xt)

    return jnp.transpose(out.reshape(_NOUT, B, H, W), (1, 2, 3, 0)).astype(
        jnp.float32)


# speedup vs baseline: 4.3095x; 2.8310x over previous
import functools

import jax
import jax.numpy as jnp
import numpy as np
from jax.experimental import pallas as pl
from jax.experimental.pallas import tpu as pltpu

_LOG2E = 1.4426950408889634
_NOUT = 11
_RH = 256
_RC = 16
_SIGN = np.uint32(0x80000000)


def _histo_body(k2_ref, b2_ref, x_ref, o_ref):
    def chunk(i, carry):
        rows = pl.ds(i * _RC, _RC)
        ys = [x_ref[0, c, rows, :] for c in range(3)]
        for j in range(_NOUT):
            kj = k2_ref[j]
            bj = b2_ref[j]
            acc = None
            for y in ys:
                x2 = y * kj + bj
                neg_abs = pltpu.bitcast(
                    pltpu.bitcast(x2, jnp.uint32) | _SIGN, jnp.float32)
                t = jnp.exp2(neg_abs)
                s = t * (1.0 / (1.0 + t * t))
                acc = s if acc is None else acc + s
            o_ref[0, j, rows, :] = acc + acc
        return carry

    jax.lax.fori_loop(0, _RH // _RC, chunk, 0)


@functools.partial(jax.jit, static_argnames=())
def kernel(image, kernel, bias):
    B, H, W, C = image.shape
    xt = jnp.transpose(image, (0, 3, 1, 2))
    k2 = (kernel * _LOG2E).astype(jnp.float32)
    b2 = (bias * _LOG2E).astype(jnp.float32)

    out = pl.pallas_call(
        _histo_body,
        out_shape=jax.ShapeDtypeStruct((B, _NOUT, H, W), jnp.float32),
        grid=(B, H // _RH),
        in_specs=[
            pl.BlockSpec(memory_space=pltpu.SMEM),
            pl.BlockSpec(memory_space=pltpu.SMEM),
            pl.BlockSpec((1, C, _RH, W), lambda i, j: (i, 0, j, 0)),
        ],
        out_specs=pl.BlockSpec((1, _NOUT, _RH, W), lambda i, j: (i, 0, j, 0)),
        compiler_params=pltpu.CompilerParams(
            dimension_semantics=("arbitrary", "arbitrary"),
            vmem_limit_bytes=56 * 1024 * 1024,
        ),
        name="histo_sech",
    )(k2, b2, xt)

    return jnp.transpose(out, (0, 2, 3, 1))


# speedup vs baseline: 5.1008x; 1.1836x over previous
import functools

import jax
import jax.numpy as jnp
import numpy as np
from jax.experimental import pallas as pl
from jax.experimental.pallas import tpu as pltpu

_LOG2E = 1.4426950408889634
_NOUT = 11
_RH = 256
_RC = 256
_SIGN16 = np.uint16(0x8000)


def _histo_body(k2_ref, b2_ref, x_ref, o_ref):
    def chunk(i, carry):
        rows = pl.ds(i * _RC, _RC)
        ys = [x_ref[0, c, rows, :].astype(jnp.bfloat16) for c in range(3)]
        for j in range(_NOUT):
            kj = k2_ref[j].astype(jnp.bfloat16)
            bj = b2_ref[j].astype(jnp.bfloat16)
            acc = None
            for y in ys:
                x2 = y * kj + bj
                neg_abs = pltpu.bitcast(
                    pltpu.bitcast(x2, jnp.uint16) | _SIGN16, jnp.bfloat16)
                t = jnp.exp2(neg_abs)
                s = t / (1.0 + t * t)
                acc = s if acc is None else acc + s
            o_ref[0, j, rows, :] = (acc + acc).astype(jnp.float32)
        return carry

    jax.lax.fori_loop(0, _RH // _RC, chunk, 0)


@functools.partial(jax.jit, static_argnames=())
def kernel(image, kernel, bias):
    B, H, W, C = image.shape
    xt = jnp.transpose(image, (0, 3, 1, 2))
    k2 = (kernel * _LOG2E).astype(jnp.float32)
    b2 = (bias * _LOG2E).astype(jnp.float32)

    out = pl.pallas_call(
        _histo_body,
        out_shape=jax.ShapeDtypeStruct((B, _NOUT, H, W), jnp.float32),
        grid=(B, H // _RH),
        in_specs=[
            pl.BlockSpec(memory_space=pltpu.SMEM),
            pl.BlockSpec(memory_space=pltpu.SMEM),
            pl.BlockSpec((1, C, _RH, W), lambda i, j: (i, 0, j, 0)),
        ],
        out_specs=pl.BlockSpec((1, _NOUT, _RH, W), lambda i, j: (i, 0, j, 0)),
        compiler_params=pltpu.CompilerParams(
            dimension_semantics=("arbitrary", "arbitrary"),
            vmem_limit_bytes=56 * 1024 * 1024,
        ),
        name="histo_sech",
    )(k2, b2, xt)

    return jnp.transpose(out, (0, 2, 3, 1))
